# baseline (speedup 1.0000x reference)
import functools

import jax
import jax.numpy as jnp
from jax import lax
from jax.experimental import pallas as pl
from jax.experimental.pallas import tpu as pltpu
from jax.experimental.pallas import tpu_sc as plsc

ROWS = 128
N = 32768
L = 16
CHUNKS = N // L
BLK = 16
NB = CHUNKS // BLK
NG = NB // 16
NUM_WORKERS = 32
ROWS_PER_WORKER = ROWS // NUM_WORKERS
BISECT_ITERS = 13
UNROLL = 8
NEG_BIG = -3e38

_mesh = plsc.VectorSubcoreMesh(core_axis_name="c", subcore_axis_name="s")


def _bfly_max(v, lane):
    for sh in (1, 2, 4, 8):
        v = jnp.maximum(v, v[lane ^ sh])
    return v


def _bfly_sum(v, lane):
    for sh in (1, 2, 4, 8):
        v = v + v[lane ^ sh]
    return v


def _htree_max(regs, lane):
    level = list(regs)
    for k in (1, 2, 4, 8):
        clear = (lane & k) == 0
        nxt = []
        for i in range(0, len(level), 2):
            a, b = level[i], level[i + 1]
            s = jnp.where(clear, a, b)
            u = jnp.where(clear, b, a)
            nxt.append(jnp.maximum(s, u[lane ^ k]))
        level = nxt
    return level[0]


def _process_row(row_v, cand_v, bmax_v, blist_s, lane, fixup_out):
    neg = jnp.full((L,), NEG_BIG, jnp.float32)

    @plsc.parallel_loop(0, NB, carry=(neg,) * 8)
    def gmax8(b, gaccs):
        base = b * (BLK * L)
        cs = [row_v[pl.ds(base + u * L, L)] for u in range(BLK)]
        m = [jnp.maximum(cs[2 * u], cs[2 * u + 1]) for u in range(8)]
        bm = m[0]
        for u in range(1, 8):
            bm = jnp.maximum(bm, m[u])
        bmax_v[pl.ds(b * L, L)] = bm
        return tuple(jnp.maximum(gaccs[u], m[u]) for u in range(8))

    gmax = gmax8[0]
    for u in range(1, 8):
        gmax = jnp.maximum(gmax, gmax8[u])
    mx = _bfly_max(gmax, lane)[0]
    lo0 = mx - 1.0

    def p2a_body(g, nb):
        regs = [bmax_v[pl.ds((g * 16 + t) * L, L)] for t in range(16)]
        bm = _htree_max(regs, lane)
        for t in range(16):
            blist_s[nb] = g * 16 + t
            nb = nb + (bm[t] > lo0).astype(jnp.int32)
        return nb

    nb = lax.fori_loop(0, NG, p2a_body, jnp.int32(0))

    def p2b_body(i, cnt):
        b = blist_s[i]
        base = b * (BLK * L)
        regs = [row_v[pl.ds(base + t * L, L)] for t in range(16)]
        cm = _htree_max(regs, lane)
        for t in range(16):
            cand_v[pl.ds(cnt * L, L)] = regs[t]
            cnt = cnt + (cm[t] > lo0).astype(jnp.int32)
        return cnt

    nch = lax.fori_loop(0, nb, p2b_body, jnp.int32(0))

    for u in range(8):
        cand_v[pl.ds((nch + u) * L, L)] = neg

    ones = jnp.full((L,), 1.0, jnp.float32)
    zero = jnp.zeros((L,), jnp.float32)
    lo0v = zero + lo0
    mxv = zero + mx

    def bis_body(_, lh):
        lov, hiv = lh
        midv = 0.5 * (lov + hiv)

        @plsc.parallel_loop(0, nch, step=4, carry=(zero,) * 4)
        def acc4(i, accs):
            return tuple(
                accs[u] + jnp.maximum(cand_v[pl.ds((i + u) * L, L)] - midv,
                                      0.0)
                for u in range(4))

        acc = (acc4[0] + acc4[1]) + (acc4[2] + acc4[3])
        gt = _bfly_sum(acc, lane) > ones
        return jnp.where(gt, midv, lov), jnp.where(gt, hiv, midv)

    lov, _hiv = lax.fori_loop(0, BISECT_ITERS, bis_body, (lo0v, mxv))

    def michelot(tv):
        @plsc.parallel_loop(0, nch, step=2, carry=(zero, zero, zero, zero))
        def ex_carry(i, carry):
            s0, k0, s1, k1 = carry
            out = []
            for u, (s, k) in enumerate(((s0, k0), (s1, k1))):
                v = cand_v[pl.ds((i + u) * L, L)]
                msk = v > tv
                out.append(s + jnp.where(msk, v, zero))
                out.append(k + jnp.where(msk, ones, zero))
            return tuple(out)

        sv = ex_carry[0] + ex_carry[2]
        kv = ex_carry[1] + ex_carry[3]
        return (_bfly_sum(sv, lane) - 1.0) / _bfly_sum(kv, lane)

    tauv = michelot(michelot(michelot(lov)))
    tau = tauv[0]

    fixup_out(nb, tau)
    return nb


ZQ = 4
ZN = N // ZQ


@functools.partial(
    pl.kernel,
    out_type=jax.ShapeDtypeStruct((ROWS, N), jnp.float32),
    mesh=_mesh,
    scratch_types=[
        pltpu.VMEM((N,), jnp.float32),
        pltpu.VMEM((N,), jnp.float32),
        pltpu.VMEM((N + 8 * L,), jnp.float32),
        pltpu.VMEM((NB * L,), jnp.float32),
        pltpu.VMEM((ZN,), jnp.float32),
        pltpu.SMEM((NB,), jnp.int32),
        pltpu.SemaphoreType.DMA,
        pltpu.SemaphoreType.DMA,
        pltpu.SemaphoreType.DMA,
        pltpu.SemaphoreType.DMA,
    ],
)
def _sparsemax_sc(x_hbm, out_hbm, row_a, row_b, cand_v, bmax_v, zero_v,
                  blist_s, si_a, si_b, sz, sf):
    wid = lax.axis_index("s") * 2 + lax.axis_index("c")
    lane = lax.iota(jnp.int32, L)
    bufs = [(row_a, si_a), (row_b, si_b)]
    base_row = wid * ROWS_PER_WORKER

    zero = jnp.zeros((L,), jnp.float32)

    @plsc.parallel_loop(0, ZN // L, unroll=UNROLL)
    def _zinit(i):
        zero_v[pl.ds(i * L, L)] = zero

    pltpu.make_async_copy(x_hbm.at[base_row], row_a, si_a).start()
    prev_nb = None
    for j in range(ROWS_PER_WORKER):
        x_v, si = bufs[j % 2]
        row = base_row + j
        out_row = out_hbm.at[row]

        for q in range(ZQ):
            pltpu.make_async_copy(zero_v, out_row.at[pl.ds(q * ZN, ZN)],
                                  sz).start()
        if j + 1 < ROWS_PER_WORKER:
            y_v, si_y = bufs[(j + 1) % 2]
            pltpu.make_async_copy(x_hbm.at[row + 1], y_v, si_y).start()

        pltpu.make_async_copy(x_hbm.at[row], x_v, si).wait()

        def fixup_out(nb, tau, x_v=x_v, out_row=out_row, j=j):
            for q in range(ZQ):
                pltpu.make_async_copy(zero_v, out_row.at[pl.ds(q * ZN, ZN)],
                                      sz).wait()

            def fb_body(i, carry):
                b = blist_s[i]
                base = b * (BLK * L)
                sbase = i * (BLK * L)
                for t in range(BLK):
                    v = x_v[pl.ds(base + t * L, L)]
                    cand_v[pl.ds(sbase + t * L, L)] = jnp.maximum(v - tau, 0.0)
                pltpu.make_async_copy(
                    cand_v.at[pl.ds(sbase, BLK * L)],
                    out_row.at[pl.ds(base, BLK * L)], sf).start()
                return carry

            lax.fori_loop(0, nb, fb_body, jnp.int32(0))

        if prev_nb is not None:
            prev_out = out_hbm.at[row - 1]

            def drain_body(i, carry, prev_out=prev_out):
                pltpu.make_async_copy(
                    cand_v.at[pl.ds(i * (BLK * L), BLK * L)],
                    prev_out.at[pl.ds(i * (BLK * L), BLK * L)], sf).wait()
                return carry

            lax.fori_loop(0, prev_nb, drain_body, jnp.int32(0))

        prev_nb = _process_row(x_v, cand_v, bmax_v, blist_s, lane, fixup_out)

    def drain_last(i, carry):
        pltpu.make_async_copy(
            cand_v.at[pl.ds(i * (BLK * L), BLK * L)],
            out_hbm.at[base_row + ROWS_PER_WORKER - 1].at[
                pl.ds(i * (BLK * L), BLK * L)], sf).wait()
        return carry

    lax.fori_loop(0, prev_nb, drain_last, jnp.int32(0))


def kernel(inputs):
    return _sparsemax_sc(inputs)


# speedup vs baseline: 1.3101x; 1.3101x over previous
import functools

import jax
import jax.numpy as jnp
from jax import lax
from jax.experimental import pallas as pl
from jax.experimental.pallas import tpu as pltpu
from jax.experimental.pallas import tpu_sc as plsc

ROWS = 128
N = 32768
L = 16
CHUNKS = N // L
BLK = 16
NB = CHUNKS // BLK
NG = NB // 16
NUM_WORKERS = 32
ROWS_PER_WORKER = ROWS // NUM_WORKERS
BISECT_ITERS = 13
UNROLL = 8
NEG_BIG = -3e38

_mesh = plsc.VectorSubcoreMesh(core_axis_name="c", subcore_axis_name="s")


def _bfly_max(v, lane):
    for sh in (1, 2, 4, 8):
        v = jnp.maximum(v, v[lane ^ sh])
    return v


def _bfly_sum(v, lane):
    for sh in (1, 2, 4, 8):
        v = v + v[lane ^ sh]
    return v


def _htree_max(regs, lane):
    level = list(regs)
    for k in (1, 2, 4, 8):
        clear = (lane & k) == 0
        nxt = []
        for i in range(0, len(level), 2):
            a, b = level[i], level[i + 1]
            s = jnp.where(clear, a, b)
            u = jnp.where(clear, b, a)
            nxt.append(jnp.maximum(s, u[lane ^ k]))
        level = nxt
    return level[0]


def _process_row(row_v, cand_v, bmax_v, blist_s, lane, prefetch):
    neg = jnp.full((L,), NEG_BIG, jnp.float32)

    @plsc.parallel_loop(0, NB, carry=(neg,) * 8)
    def gmax8(b, gaccs):
        base = b * (BLK * L)
        cs = [row_v[pl.ds(base + u * L, L)] for u in range(BLK)]
        m = [jnp.maximum(cs[2 * u], cs[2 * u + 1]) for u in range(8)]
        bm = m[0]
        for u in range(1, 8):
            bm = jnp.maximum(bm, m[u])
        bmax_v[pl.ds(b * L, L)] = bm
        return tuple(jnp.maximum(gaccs[u], m[u]) for u in range(8))

    gmax = gmax8[0]
    for u in range(1, 8):
        gmax = jnp.maximum(gmax, gmax8[u])
    mx = _bfly_max(gmax, lane)[0]
    lo0 = mx - 1.0

    def p2a_body(g, nb):
        regs = [bmax_v[pl.ds((g * 16 + t) * L, L)] for t in range(16)]
        bm = _htree_max(regs, lane)
        for t in range(16):
            blist_s[nb] = g * 16 + t
            nb = nb + (bm[t] > lo0).astype(jnp.int32)
        return nb

    nb = lax.fori_loop(0, NG, p2a_body, jnp.int32(0))

    def p2b_body(i, cnt):
        b = blist_s[i]
        base = b * (BLK * L)
        regs = [row_v[pl.ds(base + t * L, L)] for t in range(16)]
        cm = _htree_max(regs, lane)
        for t in range(16):
            cand_v[pl.ds(cnt * L, L)] = regs[t]
            cnt = cnt + (cm[t] > lo0).astype(jnp.int32)
        return cnt

    nch = lax.fori_loop(0, nb, p2b_body, jnp.int32(0))

    for u in range(8):
        cand_v[pl.ds((nch + u) * L, L)] = neg

    prefetch()

    ones = jnp.full((L,), 1.0, jnp.float32)
    zero = jnp.zeros((L,), jnp.float32)
    lo0v = zero + lo0
    mxv = zero + mx

    def bis_body(_, lh):
        lov, hiv = lh
        midv = 0.5 * (lov + hiv)

        @plsc.parallel_loop(0, nch, step=8, carry=(zero,) * 8)
        def acc8(i, accs):
            return tuple(
                accs[u] + jnp.maximum(cand_v[pl.ds((i + u) * L, L)] - midv,
                                      0.0)
                for u in range(8))

        acc = ((acc8[0] + acc8[1]) + (acc8[2] + acc8[3]) +
               ((acc8[4] + acc8[5]) + (acc8[6] + acc8[7])))
        gt = _bfly_sum(acc, lane) > ones
        return jnp.where(gt, midv, lov), jnp.where(gt, hiv, midv)

    lov, _hiv = lax.fori_loop(0, BISECT_ITERS, bis_body, (lo0v, mxv))

    def michelot(tv):
        @plsc.parallel_loop(0, nch, step=2, carry=(zero, zero, zero, zero))
        def ex_carry(i, carry):
            s0, k0, s1, k1 = carry
            out = []
            for u, (s, k) in enumerate(((s0, k0), (s1, k1))):
                v = cand_v[pl.ds((i + u) * L, L)]
                msk = v > tv
                out.append(s + jnp.where(msk, v, zero))
                out.append(k + jnp.where(msk, ones, zero))
            return tuple(out)

        sv = ex_carry[0] + ex_carry[2]
        kv = ex_carry[1] + ex_carry[3]
        return (_bfly_sum(sv, lane) - 1.0) / _bfly_sum(kv, lane)

    tauv = michelot(michelot(michelot(lov)))
    tau = tauv[0]

    @plsc.parallel_loop(0, CHUNKS, unroll=UNROLL)
    def _p3(i):
        v = row_v[pl.ds(i * L, L)]
        row_v[pl.ds(i * L, L)] = jnp.maximum(v - tau, 0.0)


@functools.partial(
    pl.kernel,
    out_type=jax.ShapeDtypeStruct((ROWS, N), jnp.float32),
    mesh=_mesh,
    scratch_types=[
        pltpu.VMEM((N,), jnp.float32),
        pltpu.VMEM((N,), jnp.float32),
        pltpu.VMEM((N + 8 * L,), jnp.float32),
        pltpu.VMEM((NB * L,), jnp.float32),
        pltpu.SMEM((NB,), jnp.int32),
        pltpu.SemaphoreType.DMA,
        pltpu.SemaphoreType.DMA,
        pltpu.SemaphoreType.DMA,
        pltpu.SemaphoreType.DMA,
    ],
)
def _sparsemax_sc(x_hbm, out_hbm, row_a, row_b, cand_v, bmax_v, blist_s,
                  si_a, si_b, so_a, so_b):
    wid = lax.axis_index("s") * 2 + lax.axis_index("c")
    lane = lax.iota(jnp.int32, L)
    bufs = [(row_a, si_a, so_a), (row_b, si_b, so_b)]
    base_row = wid * ROWS_PER_WORKER

    pltpu.make_async_copy(x_hbm.at[base_row], row_a, si_a).start()
    for j in range(ROWS_PER_WORKER):
        x_v, si, so = bufs[j % 2]
        row = base_row + j
        pltpu.make_async_copy(x_hbm.at[row], x_v, si).wait()

        def prefetch(j=j, row=row):
            if j + 1 < ROWS_PER_WORKER:
                y_v, si_y, so_y = bufs[(j + 1) % 2]
                if j >= 1:
                    pltpu.make_async_copy(y_v, out_hbm.at[row - 1],
                                          so_y).wait()
                pltpu.make_async_copy(x_hbm.at[row + 1], y_v, si_y).start()

        _process_row(x_v, cand_v, bmax_v, blist_s, lane, prefetch)
        pltpu.make_async_copy(x_v, out_hbm.at[row], so).start()

    pltpu.make_async_copy(row_a, out_hbm.at[base_row + 2], so_a).wait()
    pltpu.make_async_copy(row_b, out_hbm.at[base_row + 3], so_b).wait()


def kernel(inputs):
    return _sparsemax_sc(inputs)
